# Initial kernel scaffold; baseline (speedup 1.0000x reference)
import jax
import jax.numpy as jnp
from jax.experimental import pallas as pl


def kernel(x, adj, rng, pool_w_rel1, pool_b1, pool_w_root1, pool_w_rel2, pool_b2, pool_w_root2, pool_w_rel3, pool_b3, pool_w_root3, pool_bn1_w, pool_bn1_b, pool_bn2_w, pool_bn2_b, pool_w_lin, pool_b_lin, emb_w_rel1, emb_b1, emb_w_root1, emb_w_rel2, emb_b2, emb_w_root2, emb_w_rel3, emb_b3, emb_w_root3, emb_bn1_w, emb_bn1_b, emb_bn2_w, emb_bn2_b):
    raise NotImplementedError("write your pallas kernel here")


# speedup vs baseline: 1.0008x; 1.0008x over previous
import jax
import jax.numpy as jnp
from jax import lax
from jax.experimental import pallas as pl
from jax.experimental.pallas import tpu as pltpu

_BN_EPS = 1e-5
_NORM_EPS = 1e-12
_DIFFPOOL_EPS = 1e-15
_VMEM_LIMIT = 48 * 1024 * 1024


def _inv_deg(adj):
    return 1.0 / jnp.maximum(jnp.sum(adj, axis=-1, keepdims=True), 1.0)


def _l2norm(out):
    ss = jnp.sum(out * out, axis=-1, keepdims=True)
    return out * lax.rsqrt(jnp.maximum(ss, _NORM_EPS * _NORM_EPS))


def _stats(a, b, h):
    z = jnp.zeros((4, a.shape[1]), jnp.float32)
    return jnp.concatenate([
        jnp.sum(a, axis=0, keepdims=True),
        jnp.sum(a * a, axis=0, keepdims=True),
        jnp.sum(b, axis=0, keepdims=True),
        jnp.sum(b * b, axis=0, keepdims=True), z], axis=0)


def _stage1_body(x_ref, adj_ref, w1p_ref, w1e_ref, vec_ref,
                 r1p_ref, r1e_ref, st_ref):
    adj = adj_ref[...]
    x = x_ref[...]
    agg = jnp.dot(adj, x, preferred_element_type=jnp.float32) * _inv_deg(adj)
    cat = jnp.concatenate([agg, x], axis=-1)
    vec = vec_ref[...]

    def sage_relu(w_ref, b):
        out = jnp.dot(cat, w_ref[...], preferred_element_type=jnp.float32) + b
        return jnp.maximum(_l2norm(out), 0.0)

    r1p = sage_relu(w1p_ref, vec[0:1])
    r1e = sage_relu(w1e_ref, vec[1:2])
    r1p_ref[...] = r1p
    r1e_ref[...] = r1e
    st_ref[...] = _stats(r1p, r1e, r1p.shape[1])


def _stage2_body(r1p_ref, r1e_ref, adj_ref, w2p_ref, w2e_ref, vec_ref,
                 h1p_ref, h1e_ref, r2p_ref, r2e_ref, st_ref):
    adj = adj_ref[...]
    vec = vec_ref[...]
    h1p = (r1p_ref[...] - vec[0:1]) * vec[1:2] + vec[2:3]
    h1e = (r1e_ref[...] - vec[3:4]) * vec[4:5] + vec[5:6]
    agg = jnp.dot(adj, jnp.concatenate([h1p, h1e], axis=-1),
                  preferred_element_type=jnp.float32) * _inv_deg(adj)
    H = h1p.shape[1]

    def sage_relu(h, a, w_ref, b):
        out = jnp.dot(jnp.concatenate([a, h], axis=-1), w_ref[...],
                      preferred_element_type=jnp.float32) + b
        return jnp.maximum(_l2norm(out), 0.0)

    r2p = sage_relu(h1p, agg[:, :H], w2p_ref, vec[6:7])
    r2e = sage_relu(h1e, agg[:, H:], w2e_ref, vec[7:8])
    h1p_ref[...] = h1p
    h1e_ref[...] = h1e
    r2p_ref[...] = r2p
    r2e_ref[...] = r2e
    st_ref[...] = _stats(r2p, r2e, H)


def _stage3_body(h1p_ref, h1e_ref, r2p_ref, r2e_ref, adj_ref,
                 w3p_ref, w3e_ref, wlin_ref, vec_ref, gd_ref,
                 out_x_ref, out_adj_ref, s_ref, link_ref, ent_ref):
    adj = adj_ref[...]
    vec = vec_ref[...]
    h2p = (r2p_ref[...] - vec[0:1]) * vec[1:2] + vec[2:3]
    h2e = (r2e_ref[...] - vec[3:4]) * vec[4:5] + vec[5:6]
    agg = jnp.dot(adj, jnp.concatenate([h2p, h2e], axis=-1),
                  preferred_element_type=jnp.float32) * _inv_deg(adj)
    H = h2p.shape[1]

    def sage(h, a, w_ref, b):
        out = jnp.dot(jnp.concatenate([a, h], axis=-1), w_ref[...],
                      preferred_element_type=jnp.float32) + b
        return _l2norm(out)

    h3p = sage(h2p, agg[:, :H], w3p_ref, vec[6:7])
    h3e = sage(h2e, agg[:, H:], w3e_ref, vec[7:8])

    logits = (jnp.dot(jnp.concatenate([h1p_ref[...], h2p, h3p], axis=-1),
                      wlin_ref[...], preferred_element_type=jnp.float32)
              + vec[8:9])

    m = jnp.max(logits, axis=-1, keepdims=True)
    e = jnp.exp(logits - m)
    sb = e / jnp.sum(e, axis=-1, keepdims=True)
    s_ref[...] = sb

    xb = jnp.concatenate([h1e_ref[...], h2e, h3e], axis=-1)
    cT = (((0,), (0,)), ((), ()))
    out_x_ref[...] = lax.dot_general(sb, xb, cT,
                                     preferred_element_type=jnp.float32)
    sta = lax.dot_general(sb, adj, cT, preferred_element_type=jnp.float32)
    pooled = jnp.dot(sta, sb, preferred_element_type=jnp.float32)
    sts = lax.dot_general(sb, sb, cT, preferred_element_type=jnp.float32)

    K = sb.shape[1]
    row = lax.broadcasted_iota(jnp.int32, (K, K), 0)
    col = lax.broadcasted_iota(jnp.int32, (K, K), 1)
    diag = row == col

    sum_adj2 = jnp.sum(jnp.sum(adj * adj, axis=1, keepdims=True),
                       axis=0, keepdims=True)
    tr_pooled = jnp.sum(jnp.sum(jnp.where(diag, pooled, 0.0),
                                axis=1, keepdims=True),
                        axis=0, keepdims=True)
    sum_sts2 = jnp.sum(jnp.sum(sts * sts, axis=1, keepdims=True),
                       axis=0, keepdims=True)
    link_ref[...] = sum_adj2 - 2.0 * tr_pooled + sum_sts2

    ent = -sb * jnp.log(sb + _DIFFPOOL_EPS)
    ent_ref[...] = jnp.sum(jnp.sum(ent, axis=1, keepdims=True),
                           axis=0, keepdims=True)

    mn = jnp.min(jnp.min(pooled, axis=1, keepdims=True), axis=0, keepdims=True)
    mx = jnp.max(jnp.max(pooled, axis=1, keepdims=True), axis=0, keepdims=True)
    an = (pooled - mn) / jnp.maximum(mx - mn, 1e-12)
    hard = jnp.where(an + gd_ref[...] >= 1.0 - an, 1.0, 0.0)
    ut = jnp.where(col >= row, hard, 0.0)
    sym = ut + ut.T
    out_adj_ref[...] = jnp.where(diag, 1.0, sym)


def _full(shape):
    return pl.BlockSpec(shape, lambda b: (0,) * len(shape))


def _bat(*shape):
    return pl.BlockSpec((None,) + shape, lambda b: (b,) + (0,) * len(shape))


def _params(n):
    return pltpu.CompilerParams(
        dimension_semantics=("parallel",) * n,
        vmem_limit_bytes=_VMEM_LIMIT)


def kernel(x, adj, rng, pool_w_rel1, pool_b1, pool_w_root1, pool_w_rel2,
           pool_b2, pool_w_root2, pool_w_rel3, pool_b3, pool_w_root3,
           pool_bn1_w, pool_bn1_b, pool_bn2_w, pool_bn2_b, pool_w_lin,
           pool_b_lin, emb_w_rel1, emb_b1, emb_w_root1, emb_w_rel2, emb_b2,
           emb_w_root2, emb_w_rel3, emb_b3, emb_w_root3, emb_bn1_w,
           emb_bn1_b, emb_bn2_w, emb_bn2_b):
    B, N, C = x.shape
    H = pool_w_rel1.shape[1]
    K = pool_w_lin.shape[1]
    Fe = emb_w_rel3.shape[1]
    D = 2 * H + Fe
    inv_bn = 1.0 / float(B * N)

    key = jax.random.wrap_key_data(rng)
    g = jax.random.gumbel(key, (2, B, K, K), jnp.float32)
    gd = g[0] - g[1]

    def wcat(wr, wo):
        return jnp.concatenate([wr, wo], axis=0)

    w1p = wcat(pool_w_rel1, pool_w_root1)
    w2p = wcat(pool_w_rel2, pool_w_root2)
    w3p = wcat(pool_w_rel3, pool_w_root3)
    w1e = wcat(emb_w_rel1, emb_w_root1)
    w2e = wcat(emb_w_rel2, emb_w_root2)
    w3e = wcat(emb_w_rel3, emb_w_root3)

    zrow = jnp.zeros((1, H), jnp.float32)
    vec1 = jnp.concatenate([pool_b1, emb_b1] + [zrow] * 6, axis=0)

    r1p, r1e, st1 = pl.pallas_call(
        _stage1_body,
        grid=(B,),
        in_specs=[_bat(N, C), _bat(N, N),
                  _full((2 * C, H)), _full((2 * C, H)), _full((8, H))],
        out_specs=(_bat(N, H), _bat(N, H), _bat(8, H)),
        out_shape=(jax.ShapeDtypeStruct((B, N, H), jnp.float32),
                   jax.ShapeDtypeStruct((B, N, H), jnp.float32),
                   jax.ShapeDtypeStruct((B, 8, H), jnp.float32)),
        compiler_params=_params(1),
    )(x, adj, w1p, w1e, vec1)

    def bn_rows(st):
        s = jnp.sum(st, axis=0)
        mean_p = s[0:1] * inv_bn
        var_p = jnp.maximum(s[1:2] * inv_bn - mean_p * mean_p, 0.0)
        mean_e = s[2:3] * inv_bn
        var_e = jnp.maximum(s[3:4] * inv_bn - mean_e * mean_e, 0.0)
        return mean_p, var_p, mean_e, var_e

    mp, vp, me, ve = bn_rows(st1)
    vec2 = jnp.concatenate([
        mp, lax.rsqrt(vp + _BN_EPS) * pool_bn1_w, pool_bn1_b,
        me, lax.rsqrt(ve + _BN_EPS) * emb_bn1_w, emb_bn1_b,
        pool_b2, emb_b2], axis=0)

    h1p, h1e, r2p, r2e, st2 = pl.pallas_call(
        _stage2_body,
        grid=(B,),
        in_specs=[_bat(N, H), _bat(N, H), _bat(N, N),
                  _full((2 * H, H)), _full((2 * H, H)), _full((8, H))],
        out_specs=(_bat(N, H), _bat(N, H), _bat(N, H), _bat(N, H),
                   _bat(8, H)),
        out_shape=(jax.ShapeDtypeStruct((B, N, H), jnp.float32),) * 4
        + (jax.ShapeDtypeStruct((B, 8, H), jnp.float32),),
        compiler_params=_params(1),
    )(r1p, r1e, adj, w2p, w2e, vec2)

    mp, vp, me, ve = bn_rows(st2)
    vec3 = jnp.concatenate([
        mp, lax.rsqrt(vp + _BN_EPS) * pool_bn2_w, pool_bn2_b,
        me, lax.rsqrt(ve + _BN_EPS) * emb_bn2_w, emb_bn2_b,
        pool_b3, emb_b3, pool_b_lin] + [zrow] * 7, axis=0)

    out_x, new_adj, s_soft, link_p, ent_p = pl.pallas_call(
        _stage3_body,
        grid=(B,),
        in_specs=[_bat(N, H), _bat(N, H), _bat(N, H), _bat(N, H),
                  _bat(N, N), _full((2 * H, H)), _full((2 * H, H)),
                  _full((2 * H + K, K)), _full((16, H)), _bat(K, K)],
        out_specs=(_bat(K, D), _bat(K, K), _bat(N, K), _bat(1, 1),
                   _bat(1, 1)),
        out_shape=(jax.ShapeDtypeStruct((B, K, D), jnp.float32),
                   jax.ShapeDtypeStruct((B, K, K), jnp.float32),
                   jax.ShapeDtypeStruct((B, N, K), jnp.float32),
                   jax.ShapeDtypeStruct((B, 1, 1), jnp.float32),
                   jax.ShapeDtypeStruct((B, 1, 1), jnp.float32)),
        compiler_params=_params(1),
    )(h1p, h1e, r2p, r2e, adj, w3p, w3e, pool_w_lin, vec3, gd)

    link = jnp.sqrt(jnp.maximum(jnp.sum(link_p), 0.0)) / float(B * N * N)
    ent = jnp.sum(ent_p) / float(B * N)
    return out_x, new_adj, link, ent, s_soft
